# baseline (speedup 1.0000x reference)
import functools

import jax
import jax.numpy as jnp
from jax import lax
from jax.experimental import pallas as pl
from jax.experimental.pallas import tpu as pltpu
from jax.experimental.pallas import tpu_sc as plsc

N_NODES = 100000
D_FEAT = 128
BATCH = 50000
LANES = 16

NC, NS = 2, 16
NW = NC * NS
NBUF = 4


def _mean_agg_kernel(nchunks, nb, s, features_hbm, idx_hbm, out_hbm,
                     idx_v, rows_v, out_v, gsems, osems):
    wid = lax.axis_index("s") * NC + lax.axis_index("c")
    pltpu.sync_copy(idx_hbm.at[wid], idx_v)

    inv_s = jnp.float32(1.0 / s)
    out_base = wid * nchunks * nb
    iota = lax.iota(jnp.int32, LANES)

    for par in range(NBUF):
        pltpu.async_copy(features_hbm.at[idx_v.at[par]], rows_v.at[par],
                         gsems[par])

    def body(i, carry):
        cb = i * NBUF
        for par in range(NBUF):
            c = cb + par
            pltpu.make_async_copy(features_hbm.at[idx_v.at[c]],
                                  rows_v.at[par], gsems[par]).wait()
            @pl.when(c >= NBUF)
            def _():
                pltpu.make_async_copy(out_v.at[par],
                                      out_hbm.at[pl.ds(0, nb)],
                                      osems[par]).wait()
            out_p = out_v.at[par]
            hi_mask = jnp.full((LANES,), -65536, jnp.int32)
            for r in range(nb):
                row_idx = iota * 0 + r
                for d in range(D_FEAT // (2 * LANES)):
                    sl = pl.ds(d * LANES, LANES)

                    def halves(j):
                        v = rows_v[par, r * s + j, sl]
                        even = plsc.bitcast(v << 16, jnp.float32)
                        odd = plsc.bitcast(v & hi_mask, jnp.float32)
                        return even, odd

                    a, b_ = halves(0)
                    for j in range(1, s):
                        aj, bj = halves(j)
                        a = a + aj
                        b_ = b_ + bj
                    cols = iota * 2 + (d * 2 * LANES)
                    plsc.store_scatter(out_p, [row_idx, cols], a * inv_s)
                    plsc.store_scatter(out_p, [row_idx, cols + 1], b_ * inv_s)
            @pl.when(c + NBUF < nchunks)
            def _():
                pltpu.async_copy(features_hbm.at[idx_v.at[c + NBUF]],
                                 rows_v.at[par], gsems[par])
            pltpu.async_copy(out_v.at[par],
                             out_hbm.at[pl.ds(out_base + c * nb, nb)],
                             osems[par])
        return carry

    lax.fori_loop(0, nchunks // NBUF, body, 0)

    for par in range(NBUF):
        pltpu.make_async_copy(out_v.at[par], out_hbm.at[pl.ds(0, nb)],
                              osems[par]).wait()


def kernel(features, neigh_idx, num_sample):
    del num_sample
    b, s = neigh_idx.shape
    nb = (128 // s) // 8 * 8
    assert nb >= 8
    nchunks = -(-b // (NW * nb))
    nchunks = -(-nchunks // NBUF) * NBUF
    b_pad = NW * nchunks * nb

    feats_packed = jax.lax.bitcast_convert_type(
        features.astype(jnp.bfloat16).reshape(features.shape[0],
                                              D_FEAT // 2, 2), jnp.int32)
    idx = jnp.zeros((b_pad, s), jnp.int32).at[:b].set(neigh_idx)
    idx = idx.reshape(NW, nchunks, nb * s)

    mesh = plsc.VectorSubcoreMesh(core_axis_name="c", subcore_axis_name="s",
                                  num_cores=NC, num_subcores=NS)
    out = pl.kernel(
        functools.partial(_mean_agg_kernel, nchunks, nb, s),
        out_type=jax.ShapeDtypeStruct((b_pad, D_FEAT), jnp.float32),
        mesh=mesh,
        compiler_params=pltpu.CompilerParams(needs_layout_passes=False,
                                             use_tc_tiling_on_sc=False),
        scratch_types=[
            pltpu.VMEM((nchunks, nb * s), jnp.int32),
            pltpu.VMEM((NBUF, nb * s, D_FEAT // 2), jnp.int32),
            pltpu.VMEM((NBUF, nb, D_FEAT), jnp.float32),
            [pltpu.SemaphoreType.DMA] * NBUF,
            [pltpu.SemaphoreType.DMA] * NBUF,
        ],
    )(feats_packed, idx)
    return out[:b]


# speedup vs baseline: 1.9505x; 1.9505x over previous
import functools

import jax
import jax.numpy as jnp
from jax import lax
from jax.experimental import pallas as pl
from jax.experimental.pallas import tpu as pltpu
from jax.experimental.pallas import tpu_sc as plsc

N_NODES = 100000
D_FEAT = 128
BATCH = 50000
LANES = 16

NC, NS = 2, 16
NW = NC * NS
NBUF = 2


def _mean_agg_kernel(nchunks, nb, s, features_hbm, idx_hbm, out_hbm,
                     idx_v, rows_v, out_v, gsems, osems):
    wid = lax.axis_index("s") * NC + lax.axis_index("c")
    pltpu.sync_copy(idx_hbm.at[wid], idx_v)

    inv_s = jnp.float32(1.0 / s)
    out_base = wid * nchunks * nb

    for par in range(NBUF):
        pltpu.async_copy(features_hbm.at[idx_v.at[par]], rows_v.at[par],
                         gsems[par])

    def body(i, carry):
        cb = i * NBUF
        for par in range(NBUF):
            c = cb + par
            pltpu.make_async_copy(features_hbm.at[idx_v.at[c]],
                                  rows_v.at[par], gsems[par]).wait()
            @pl.when(c >= NBUF)
            def _():
                pltpu.make_async_copy(out_v.at[par],
                                      out_hbm.at[pl.ds(0, nb)],
                                      osems[par]).wait()
            for r in range(nb):
                for d in range(D_FEAT // LANES):
                    acc = rows_v[par, r * s, pl.ds(d * LANES, LANES)]
                    for j in range(1, s):
                        acc = acc + rows_v[par, r * s + j,
                                           pl.ds(d * LANES, LANES)]
                    out_v[par, r, pl.ds(d * LANES, LANES)] = acc * inv_s
            @pl.when(c + NBUF < nchunks)
            def _():
                pltpu.async_copy(features_hbm.at[idx_v.at[c + NBUF]],
                                 rows_v.at[par], gsems[par])
            pltpu.async_copy(out_v.at[par],
                             out_hbm.at[pl.ds(out_base + c * nb, nb)],
                             osems[par])
        return carry

    lax.fori_loop(0, nchunks // NBUF, body, 0)

    for par in range(NBUF):
        pltpu.make_async_copy(out_v.at[par], out_hbm.at[pl.ds(0, nb)],
                              osems[par]).wait()


def kernel(features, neigh_idx, num_sample):
    del num_sample
    b, s = neigh_idx.shape
    nb = 16
    nchunks = -(-b // (NW * nb))
    nchunks = -(-nchunks // NBUF) * NBUF
    b_pad = NW * nchunks * nb

    idx = jnp.zeros((b_pad, s), jnp.int32).at[:b].set(neigh_idx)
    idx = idx.reshape(NW, nchunks, nb * s)

    mesh = plsc.VectorSubcoreMesh(core_axis_name="c", subcore_axis_name="s",
                                  num_cores=NC, num_subcores=NS)
    out = pl.kernel(
        functools.partial(_mean_agg_kernel, nchunks, nb, s),
        out_type=jax.ShapeDtypeStruct((b_pad, D_FEAT), jnp.float32),
        mesh=mesh,
        compiler_params=pltpu.CompilerParams(use_tc_tiling_on_sc=False),
        scratch_types=[
            pltpu.VMEM((nchunks, nb * s), jnp.int32),
            pltpu.VMEM((NBUF, nb * s, D_FEAT), jnp.float32),
            pltpu.VMEM((NBUF, nb, D_FEAT), jnp.float32),
            [pltpu.SemaphoreType.DMA] * NBUF,
            [pltpu.SemaphoreType.DMA] * NBUF,
        ],
    )(features, idx)
    return out[:b]


# speedup vs baseline: 2.3787x; 1.2195x over previous
import functools

import jax
import jax.numpy as jnp
from jax import lax
from jax.experimental import pallas as pl
from jax.experimental.pallas import tpu as pltpu
from jax.experimental.pallas import tpu_sc as plsc

N_NODES = 100000
D_FEAT = 128
BATCH = 50000
LANES = 16

NC, NS = 2, 16
NW = NC * NS
NBUF = 4


def _mean_agg_kernel(nchunks, nb, s, last_base, features_hbm, idx_hbm,
                     out_hbm, idx_v, rows_v, out_v, gsems, osems):
    wid = lax.axis_index("s") * NC + lax.axis_index("c")
    rows_per_worker = nchunks * nb
    base = jnp.minimum(wid * rows_per_worker, last_base)
    pltpu.sync_copy(idx_hbm.at[pl.ds(base * s, rows_per_worker * s)], idx_v)

    inv_s = jnp.float32(1.0 / s)

    for par in range(NBUF):
        pltpu.async_copy(
            features_hbm.at[idx_v.at[pl.ds(par * nb * s, nb * s)]],
            rows_v.at[par], gsems[par])

    def body(i, carry):
        cb = i * NBUF
        for par in range(NBUF):
            c = cb + par
            pltpu.make_async_copy(
                features_hbm.at[idx_v.at[pl.ds(c * nb * s, nb * s)]],
                rows_v.at[par], gsems[par]).wait()
            @pl.when(c >= NBUF)
            def _():
                pltpu.make_async_copy(out_v.at[par],
                                      out_hbm.at[pl.ds(0, nb)],
                                      osems[par]).wait()
            for r in range(nb):
                for d in range(D_FEAT // LANES):
                    acc = rows_v[par, r * s, pl.ds(d * LANES, LANES)]
                    for j in range(1, s):
                        acc = acc + rows_v[par, r * s + j,
                                           pl.ds(d * LANES, LANES)]
                    out_v[par, r, pl.ds(d * LANES, LANES)] = acc * inv_s
            @pl.when(c + NBUF < nchunks)
            def _():
                pltpu.async_copy(
                    features_hbm.at[
                        idx_v.at[pl.ds((c + NBUF) * nb * s, nb * s)]],
                    rows_v.at[par], gsems[par])
            pltpu.async_copy(out_v.at[par],
                             out_hbm.at[pl.ds(base + c * nb, nb)],
                             osems[par])
        return carry

    lax.fori_loop(0, nchunks // NBUF, body, 0)

    for par in range(NBUF):
        pltpu.make_async_copy(out_v.at[par], out_hbm.at[pl.ds(0, nb)],
                              osems[par]).wait()


def kernel(features, neigh_idx, num_sample):
    del num_sample
    b, s = neigh_idx.shape
    nb = (128 // s) // 8 * 8
    assert nb >= 8
    nchunks = -(-b // (NW * nb))
    nchunks = -(-nchunks // NBUF) * NBUF
    rows_per_worker = nchunks * nb
    last_base = (b - rows_per_worker) // 8 * 8
    assert last_base >= 0 and (NW - 1) * rows_per_worker + rows_per_worker >= b

    idx_flat = neigh_idx.reshape(-1).astype(jnp.int32)

    mesh = plsc.VectorSubcoreMesh(core_axis_name="c", subcore_axis_name="s",
                                  num_cores=NC, num_subcores=NS)
    out = pl.kernel(
        functools.partial(_mean_agg_kernel, nchunks, nb, s, last_base),
        out_type=jax.ShapeDtypeStruct((b, D_FEAT), jnp.float32),
        mesh=mesh,
        scratch_types=[
            pltpu.VMEM((rows_per_worker * s,), jnp.int32),
            pltpu.VMEM((NBUF, nb * s, D_FEAT), jnp.float32),
            pltpu.VMEM((NBUF, nb, D_FEAT), jnp.float32),
            [pltpu.SemaphoreType.DMA] * NBUF,
            [pltpu.SemaphoreType.DMA] * NBUF,
        ],
    )(features, idx_flat)
    return out
